# Initial kernel scaffold; baseline (speedup 1.0000x reference)
import jax
import jax.numpy as jnp
from jax.experimental import pallas as pl


def kernel(x, spiral_adj, W, b):
    raise NotImplementedError("write your pallas kernel here")


# speedup vs baseline: 2.1510x; 2.1510x over previous
import functools

import jax
import jax.numpy as jnp
from jax import lax
from jax.experimental import pallas as pl
from jax.experimental.pallas import tpu as pltpu
from jax.experimental.pallas import tpu_sc as plsc

_N = 50000
_F = 128
_S = 9
_O = 128

_MM_BLOCK = 400
_MM_GRID = _N // _MM_BLOCK

_NC = 2
_NS = 16
_NW = _NC * _NS
_CPW = 1568
_CPW_LAST = _N - (_NW - 1) * _CPW
_CH = 8
_ROWS = _CH * _S
_IDXW = _CPW * _S
_NCHUNK = _CPW // _CH
_NCHUNK_LAST = _CPW_LAST // _CH
_LANE = 16
_GROUPS = _O // _LANE


def _mm_body(x_ref, wt_ref, y_ref):
    y_ref[...] = jnp.dot(x_ref[...], wt_ref[...],
                         preferred_element_type=jnp.float32)


def _tc_matmul(x2, wt):
    return pl.pallas_call(
        _mm_body,
        grid=(_MM_GRID,),
        in_specs=[
            pl.BlockSpec((_MM_BLOCK, _F), lambda i: (i, 0)),
            pl.BlockSpec((_F, _S * _O), lambda i: (0, 0)),
        ],
        out_specs=pl.BlockSpec((_MM_BLOCK, _S * _O), lambda i: (i, 0)),
        out_shape=jax.ShapeDtypeStruct((_N, _S * _O), jnp.float32),
    )(x2, wt)


@functools.partial(
    pl.kernel,
    out_type=jax.ShapeDtypeStruct((_N, _O), jnp.float32),
    mesh=plsc.VectorSubcoreMesh(core_axis_name="c", subcore_axis_name="s"),
    scratch_types=[
        pltpu.VMEM((_IDXW,), jnp.int32),
        pltpu.VMEM((_ROWS, _O), jnp.float32),
        pltpu.VMEM((_ROWS, _O), jnp.float32),
        pltpu.VMEM((_CH, _O), jnp.float32),
        pltpu.VMEM((_O,), jnp.float32),
        pltpu.SemaphoreType.DMA,
        pltpu.SemaphoreType.DMA,
    ],
)
def _sc_gather_reduce(y_hbm, idx_hbm, b_hbm, out_hbm,
                      idx_v, rows0, rows1, outb, bias_v, sem0, sem1):
    wid = lax.axis_index("s") * _NC + lax.axis_index("c")
    node_base = wid * _CPW
    nchunks = jnp.where(wid == _NW - 1, _NCHUNK_LAST, _NCHUNK)

    pltpu.sync_copy(idx_hbm.at[pl.ds(wid * _IDXW, _IDXW)], idx_v)
    pltpu.sync_copy(b_hbm, bias_v)

    def gather(g, rows, sem):
        src = y_hbm.at[idx_v.at[pl.ds(g * _ROWS, _ROWS)]]
        return pltpu.make_async_copy(src, rows, sem)

    gather(0, rows0, sem0).start()
    gather(1, rows1, sem1).start()

    def compute(g, rows):
        for n in range(_CH):
            nid = node_base + g * _CH + n
            keep = (nid != _N - 1).astype(jnp.float32)
            for j in range(_GROUPS):
                sl = pl.ds(j * _LANE, _LANE)
                v = rows[n * _S + 0, sl]
                for s in range(1, _S):
                    v = v + rows[n * _S + s, sl]
                v = v + bias_v[sl]
                v = jnp.where(v > 0.0, v, jnp.exp(v) - 1.0)
                outb[n, sl] = v * keep
        pltpu.sync_copy(outb, out_hbm.at[pl.ds(node_base + g * _CH, _CH)])

    def body(h, carry):
        for parity, (rows, sem) in enumerate(((rows0, sem0), (rows1, sem1))):
            g = 2 * h + parity
            gather(g, rows, sem).wait()
            compute(g, rows)
            nxt = g + 2

            @pl.when(nxt < nchunks)
            def _():
                gather(nxt, rows, sem).start()
        return carry

    lax.fori_loop(0, nchunks // 2, body, 0)


def kernel(x, spiral_adj, W, b):
    B, N, F = x.shape
    S = spiral_adj.shape[-1]
    O = W.shape[0]
    assert (B, N, F, S, O) == (1, _N, _F, _S, _O)

    x2 = x.reshape(N, F)
    wt = jnp.transpose(W.reshape(O, S, F), (2, 1, 0)).reshape(F, S * O)
    y = _tc_matmul(x2, wt)
    y_rows = y.reshape(N * S, O)

    idx2 = (spiral_adj[0].astype(jnp.int32) * S
            + jnp.arange(S, dtype=jnp.int32)[None, :]).reshape(-1)
    pad = _NW * _IDXW - N * S
    idx2 = jnp.concatenate([idx2, jnp.zeros((pad,), jnp.int32)])

    out = _sc_gather_reduce(y_rows, idx2, b)
    return out.reshape(B, N, O)
